# baseline (speedup 1.0000x reference)
import jax
import jax.numpy as jnp
from jax.experimental import pallas as pl


def _fused_kernel(idx_ref, x_ref, tab_ref, w_ref, b_ref, out_ref):
    idx = idx_ref[0, :]
    onehot = (idx[:, None] == jax.lax.broadcasted_iota(
        jnp.int32, (384, 512), 1)).astype(jnp.float32)
    emb = jnp.dot(onehot, tab_ref[:, :],
                  preferred_element_type=jnp.float32)
    x = x_ref[0, :, :] + emb
    mean = jnp.mean(x, axis=-1, keepdims=True)
    xc = x - mean
    var = jnp.mean(xc * xc, axis=-1, keepdims=True)
    y = xc * jax.lax.rsqrt(var + 1e-12)
    out_ref[0, :, :] = y * w_ref[0, :] + b_ref[0, :]


def kernel(x23, idx, emb_table, ln_weight, ln_bias):
    idx = idx.astype(jnp.int32)
    out = pl.pallas_call(
        _fused_kernel,
        out_shape=jax.ShapeDtypeStruct((1, 384, 768), jnp.float32),
    )(idx, x23, emb_table, ln_weight.reshape(1, 768), ln_bias.reshape(1, 768))
    return out
